# baseline (speedup 1.0000x reference)
import functools

import jax
import jax.numpy as jnp
from jax.experimental import pallas as pl
from jax.experimental.pallas import tpu as pltpu

_T_BLK = 2048


def _batched_experts_kernel(x_ref, r_ref, w0_ref, b0_ref, w1_ref, b1_ref, o_ref):
    e = pl.program_id(1)
    h = jnp.dot(x_ref[...], w0_ref[0], preferred_element_type=jnp.float32)
    h += b0_ref[0]
    h = 0.5 * h * (1.0 + jax.lax.erf(h * 0.7071067811865476))
    y = jnp.dot(h.astype(jnp.bfloat16), w1_ref[0], preferred_element_type=jnp.float32)
    y += b1_ref[0]
    r = r_ref[...]
    col = jax.lax.broadcasted_iota(jnp.int32, r.shape, 1)
    s = jnp.sum(jnp.where(col == e, r, 0.0), axis=1, keepdims=True)
    y *= s

    @pl.when(e == 0)
    def _init():
        o_ref[...] = y

    @pl.when(e != 0)
    def _acc():
        o_ref[...] += y


@jax.jit
def kernel(x, routing_tensor, W0, b0, W1, b1):
    T, DIM = x.shape
    E = routing_tensor.shape[1]
    ED = W0.shape[2]

    xb = x.astype(jnp.bfloat16)
    W0b = W0.astype(jnp.bfloat16)
    W1b = W1.astype(jnp.bfloat16)

    grid = (T // _T_BLK, E)
    out = pl.pallas_call(
        _batched_experts_kernel,
        grid=grid,
        in_specs=[
            pl.BlockSpec((_T_BLK, DIM), lambda t, e: (t, 0)),
            pl.BlockSpec((_T_BLK, E), lambda t, e: (t, 0)),
            pl.BlockSpec((1, DIM, ED), lambda t, e: (e, 0, 0)),
            pl.BlockSpec((1, 1, ED), lambda t, e: (e, 0, 0)),
            pl.BlockSpec((1, ED, DIM), lambda t, e: (e, 0, 0)),
            pl.BlockSpec((1, 1, DIM), lambda t, e: (e, 0, 0)),
        ],
        out_specs=pl.BlockSpec((_T_BLK, DIM), lambda t, e: (t, 0)),
        out_shape=jax.ShapeDtypeStruct((T, DIM), jnp.float32),
        compiler_params=pltpu.CompilerParams(
            dimension_semantics=("parallel", "arbitrary"),
        ),
    )(xb, routing_tensor, W0b, b0, W1b, b1)
    return out


# speedup vs baseline: 1.0643x; 1.0643x over previous
import functools

import jax
import jax.numpy as jnp
from jax.experimental import pallas as pl
from jax.experimental.pallas import tpu as pltpu

_T_BLK = 512
_E_BLK = 2


def _batched_experts_kernel(x_ref, r_ref, w0_ref, b0_ref, w1_ref, b1_ref, o_ref):
    ep = pl.program_id(1)
    x = x_ref[...]
    r = r_ref[...]
    col = jax.lax.broadcasted_iota(jnp.int32, r.shape, 1)
    y = None
    for i in range(_E_BLK):
        h = jnp.dot(x, w0_ref[i], preferred_element_type=jnp.float32)
        h += b0_ref[i]
        h = 0.5 * h * (1.0 + jax.lax.erf(h * 0.7071067811865476))
        yi = jnp.dot(h.astype(jnp.bfloat16), w1_ref[i],
                     preferred_element_type=jnp.float32)
        yi += b1_ref[i]
        s = jnp.sum(jnp.where(col == ep * _E_BLK + i, r, 0.0),
                    axis=1, keepdims=True)
        yi *= s
        y = yi if y is None else y + yi

    @pl.when(ep == 0)
    def _init():
        o_ref[...] = y

    @pl.when(ep != 0)
    def _acc():
        o_ref[...] += y


@jax.jit
def kernel(x, routing_tensor, W0, b0, W1, b1):
    T, DIM = x.shape
    E = routing_tensor.shape[1]
    ED = W0.shape[2]

    xb = x.astype(jnp.bfloat16)
    W0b = W0.astype(jnp.bfloat16)
    W1b = W1.astype(jnp.bfloat16)

    grid = (T // _T_BLK, E // _E_BLK)
    out = pl.pallas_call(
        _batched_experts_kernel,
        grid=grid,
        in_specs=[
            pl.BlockSpec((_T_BLK, DIM), lambda t, e: (t, 0)),
            pl.BlockSpec((_T_BLK, E), lambda t, e: (t, 0)),
            pl.BlockSpec((_E_BLK, DIM, ED), lambda t, e: (e, 0, 0)),
            pl.BlockSpec((_E_BLK, 1, ED), lambda t, e: (e, 0, 0)),
            pl.BlockSpec((_E_BLK, ED, DIM), lambda t, e: (e, 0, 0)),
            pl.BlockSpec((_E_BLK, 1, DIM), lambda t, e: (e, 0, 0)),
        ],
        out_specs=pl.BlockSpec((_T_BLK, DIM), lambda t, e: (t, 0)),
        out_shape=jax.ShapeDtypeStruct((T, DIM), jnp.float32),
        compiler_params=pltpu.CompilerParams(
            dimension_semantics=("parallel", "arbitrary"),
        ),
    )(xb, routing_tensor, W0b, b0, W1b, b1)
    return out


# speedup vs baseline: 1.0759x; 1.0109x over previous
import functools

import jax
import jax.numpy as jnp
from jax.experimental import pallas as pl
from jax.experimental.pallas import tpu as pltpu

_T_BLK = 1024
_E_BLK = 2


def _batched_experts_kernel(x_ref, r_ref, w0_ref, b0_ref, w1_ref, b1_ref, o_ref):
    ep = pl.program_id(1)
    x = x_ref[...]
    r = r_ref[...]
    col = jax.lax.broadcasted_iota(jnp.int32, r.shape, 1)
    y = None
    for i in range(_E_BLK):
        h = jnp.dot(x, w0_ref[i], preferred_element_type=jnp.float32)
        h += b0_ref[i]
        h = 0.5 * h * (1.0 + jax.lax.erf(h * 0.7071067811865476))
        yi = jnp.dot(h.astype(jnp.bfloat16), w1_ref[i],
                     preferred_element_type=jnp.float32)
        yi += b1_ref[i]
        s = jnp.sum(jnp.where(col == ep * _E_BLK + i, r, 0.0),
                    axis=1, keepdims=True)
        yi *= s
        y = yi if y is None else y + yi

    @pl.when(ep == 0)
    def _init():
        o_ref[...] = y

    @pl.when(ep != 0)
    def _acc():
        o_ref[...] += y


@jax.jit
def kernel(x, routing_tensor, W0, b0, W1, b1):
    T, DIM = x.shape
    E = routing_tensor.shape[1]
    ED = W0.shape[2]

    xb = x.astype(jnp.bfloat16)
    W0b = W0.astype(jnp.bfloat16)
    W1b = W1.astype(jnp.bfloat16)

    grid = (T // _T_BLK, E // _E_BLK)
    out = pl.pallas_call(
        _batched_experts_kernel,
        grid=grid,
        in_specs=[
            pl.BlockSpec((_T_BLK, DIM), lambda t, e: (t, 0)),
            pl.BlockSpec((_T_BLK, E), lambda t, e: (t, 0)),
            pl.BlockSpec((_E_BLK, DIM, ED), lambda t, e: (e, 0, 0)),
            pl.BlockSpec((_E_BLK, 1, ED), lambda t, e: (e, 0, 0)),
            pl.BlockSpec((_E_BLK, ED, DIM), lambda t, e: (e, 0, 0)),
            pl.BlockSpec((_E_BLK, 1, DIM), lambda t, e: (e, 0, 0)),
        ],
        out_specs=pl.BlockSpec((_T_BLK, DIM), lambda t, e: (t, 0)),
        out_shape=jax.ShapeDtypeStruct((T, DIM), jnp.float32),
        compiler_params=pltpu.CompilerParams(
            dimension_semantics=("parallel", "arbitrary"),
        ),
    )(xb, routing_tensor, W0b, b0, W1b, b1)
    return out


# speedup vs baseline: 1.0774x; 1.0014x over previous
import functools

import jax
import jax.numpy as jnp
from jax.experimental import pallas as pl
from jax.experimental.pallas import tpu as pltpu

_T_BLK = 1024
_E_BLK = 2


def _batched_experts_kernel(x_ref, r_ref, w0_ref, b0_ref, w1_ref, b1_ref, o_ref):
    ep = pl.program_id(1)
    x = x_ref[...]
    r = r_ref[...]
    col = jax.lax.broadcasted_iota(jnp.int32, r.shape, 1)
    y = None
    for i in range(_E_BLK):
        h = jnp.dot(x, w0_ref[i], preferred_element_type=jnp.float32)
        h += b0_ref[i]
        g = (0.5 * h * (1.0 + jax.lax.erf(h * 0.7071067811865476))).astype(jnp.bfloat16)
        yi = jnp.dot(g, w1_ref[i], preferred_element_type=jnp.float32)
        yi += b1_ref[i]
        s = jnp.sum(jnp.where(col == ep * _E_BLK + i, r, 0.0),
                    axis=1, keepdims=True)
        yi *= s
        y = yi if y is None else y + yi

    @pl.when(ep == 0)
    def _init():
        o_ref[...] = y

    @pl.when(ep != 0)
    def _acc():
        o_ref[...] += y


@jax.jit
def kernel(x, routing_tensor, W0, b0, W1, b1):
    T, DIM = x.shape
    E = routing_tensor.shape[1]
    ED = W0.shape[2]

    xb = x.astype(jnp.bfloat16)
    W0b = W0.astype(jnp.bfloat16)
    W1b = W1.astype(jnp.bfloat16)

    grid = (T // _T_BLK, E // _E_BLK)
    out = pl.pallas_call(
        _batched_experts_kernel,
        grid=grid,
        in_specs=[
            pl.BlockSpec((_T_BLK, DIM), lambda t, e: (t, 0)),
            pl.BlockSpec((_T_BLK, E), lambda t, e: (t, 0)),
            pl.BlockSpec((_E_BLK, DIM, ED), lambda t, e: (e, 0, 0)),
            pl.BlockSpec((_E_BLK, 1, ED), lambda t, e: (e, 0, 0)),
            pl.BlockSpec((_E_BLK, ED, DIM), lambda t, e: (e, 0, 0)),
            pl.BlockSpec((_E_BLK, 1, DIM), lambda t, e: (e, 0, 0)),
        ],
        out_specs=pl.BlockSpec((_T_BLK, DIM), lambda t, e: (t, 0)),
        out_shape=jax.ShapeDtypeStruct((T, DIM), jnp.float32),
        compiler_params=pltpu.CompilerParams(
            dimension_semantics=("parallel", "arbitrary"),
        ),
    )(xb, routing_tensor, W0b, b0, W1b, b1)
    return out
